# baseline (speedup 1.0000x reference)
import jax
import jax.numpy as jnp
from jax.experimental import pallas as pl
from jax.experimental.pallas import tpu as pltpu


def _pick_block(n, candidates):
    for c in candidates:
        if n % c == 0:
            return c
    return n


def _gcn_body(adj_ref, mask_ref, x_ref, xi_ref, w_ref, out_ref):
    a = (adj_ref[...] * mask_ref[...]).astype(jnp.bfloat16)
    s = jnp.dot(a, x_ref[...], preferred_element_type=jnp.float32) + xi_ref[...]
    out_ref[...] = jnp.dot(s, w_ref[...].T, preferred_element_type=jnp.float32)


@jax.jit
def kernel(x, adj, mask, W):
    n, c_in = x.shape
    c_out = W.shape[0]

    bm = _pick_block(n, (200, 80, 40, 16, 8))
    grid = (n // bm,)

    x_bf = x.astype(jnp.bfloat16)

    return pl.pallas_call(
        _gcn_body,
        grid=grid,
        in_specs=[
            pl.BlockSpec((bm, n), lambda i: (i, 0)),
            pl.BlockSpec((bm, n), lambda i: (i, 0)),
            pl.BlockSpec((n, c_in), lambda i: (0, 0)),
            pl.BlockSpec((bm, c_in), lambda i: (i, 0)),
            pl.BlockSpec((c_out, c_in), lambda i: (0, 0)),
        ],
        out_specs=pl.BlockSpec((bm, c_out), lambda i: (i, 0)),
        out_shape=jax.ShapeDtypeStruct((n, c_out), jnp.float32),
        compiler_params=pltpu.CompilerParams(
            dimension_semantics=("parallel",),
        ),
    )(adj, mask, x_bf, x, W)


# speedup vs baseline: 1.0187x; 1.0187x over previous
import jax
import jax.numpy as jnp
from jax.experimental import pallas as pl
from jax.experimental.pallas import tpu as pltpu


def _pick_block(n, candidates):
    for c in candidates:
        if n % c == 0:
            return c
    return n


def _make_body(bm):
    def _gcn_body(adj_ref, mask_ref, x_ref, w_ref, out_ref, h_ref):
        i = pl.program_id(0)

        @pl.when(i == 0)
        def _compute_h():
            h_ref[...] = jnp.dot(
                x_ref[...], w_ref[...].T, preferred_element_type=jnp.float32
            ).astype(jnp.bfloat16)

        a = (adj_ref[...] * mask_ref[...]).astype(jnp.bfloat16)
        agg = jnp.dot(a, h_ref[...], preferred_element_type=jnp.float32)
        out_ref[...] = agg + h_ref[pl.ds(i * bm, bm), :].astype(jnp.float32)

    return _gcn_body


@jax.jit
def kernel(x, adj, mask, W):
    n, c_in = x.shape
    c_out = W.shape[0]

    bm = _pick_block(n, (200, 80, 40, 16, 8))
    grid = (n // bm,)

    return pl.pallas_call(
        _make_body(bm),
        grid=grid,
        in_specs=[
            pl.BlockSpec((bm, n), lambda i: (i, 0)),
            pl.BlockSpec((bm, n), lambda i: (i, 0)),
            pl.BlockSpec((n, c_in), lambda i: (0, 0)),
            pl.BlockSpec((c_out, c_in), lambda i: (0, 0)),
        ],
        out_specs=pl.BlockSpec((bm, c_out), lambda i: (i, 0)),
        out_shape=jax.ShapeDtypeStruct((n, c_out), jnp.float32),
        scratch_shapes=[pltpu.VMEM((n, c_out), jnp.bfloat16)],
        compiler_params=pltpu.CompilerParams(
            dimension_semantics=("arbitrary",),
        ),
    )(adj, mask, x, W)
